# baseline (speedup 1.0000x reference)
import jax
import jax.numpy as jnp
import numpy as np
from jax.experimental import pallas as pl
from jax.experimental.pallas import tpu as pltpu

_BN = 1024
_IN_FEATS = 392
_C1_ROWS = 9 * 512
_MC = 512


def _net_kernel(x_ref, w1_ref, b1_ref, w2_ref, b2_ref,
                fw1_ref, fb1_ref, fw2_ref, fb2_ref, out_ref, p1_ref):
    f32 = jnp.float32

    xb = x_ref[...]

    for mc in range(512 // _MC):
        lo = mc * _MC
        p1 = jnp.dot(w1_ref[lo:lo + _MC, :], xb, preferred_element_type=f32)
        for t in range(1, 9):
            c = jnp.dot(w1_ref[t * 512 + lo:t * 512 + lo + _MC, :], xb,
                        preferred_element_type=f32)
            p1 = jnp.maximum(p1, c)
        p1 = jnp.maximum(p1 + b1_ref[lo:lo + _MC, :], 0.0)
        p1_ref[lo:lo + _MC, :] = p1.astype(jnp.bfloat16)

    c2 = jnp.dot(w2_ref[...], p1_ref[...],
                 preferred_element_type=f32) + b2_ref[...]
    p2 = jnp.maximum(jnp.maximum(c2[0:64, :], c2[64:128, :]),
                     jnp.maximum(c2[128:192, :], c2[192:256, :]))
    p2 = jnp.maximum(p2, 0.0)

    h = jnp.dot(fw1_ref[...], p2.astype(jnp.bfloat16),
                preferred_element_type=f32) + fb1_ref[...]
    h = jnp.maximum(h, 0.0)

    o8 = jnp.dot(fw2_ref[...], h.astype(jnp.bfloat16),
                 preferred_element_type=f32)
    o = o8[0:1, :] + fb2_ref[...]
    out_ref[...] = jnp.tanh(jnp.maximum(o, 0.0)).astype(out_ref.dtype)


def _forward_block(xt, w1q, b1c, w2b, b2c, fw1p, fb1c, fw2p, fb2c):
    bn = _BN
    m = xt.shape[1]
    grid = (m // bn,)
    in_specs = [
        pl.BlockSpec((_IN_FEATS, bn), lambda b: (0, b)),
        pl.BlockSpec((_C1_ROWS, _IN_FEATS), lambda b: (0, 0)),
        pl.BlockSpec((512, 1), lambda b: (0, 0)),
        pl.BlockSpec((256, 512), lambda b: (0, 0)),
        pl.BlockSpec((256, 1), lambda b: (0, 0)),
        pl.BlockSpec((256, 64), lambda b: (0, 0)),
        pl.BlockSpec((256, 1), lambda b: (0, 0)),
        pl.BlockSpec((8, 256), lambda b: (0, 0)),
        pl.BlockSpec((1, 1), lambda b: (0, 0)),
    ]
    out_specs = pl.BlockSpec((1, bn), lambda b: (0, b))

    return pl.pallas_call(
        _net_kernel,
        out_shape=jax.ShapeDtypeStruct((1, m), jnp.float32),
        grid_spec=pltpu.PrefetchScalarGridSpec(
            num_scalar_prefetch=0,
            grid=grid,
            in_specs=in_specs,
            out_specs=out_specs,
            scratch_shapes=[pltpu.VMEM((512, bn), jnp.bfloat16)],
        ),
        compiler_params=pltpu.CompilerParams(
            dimension_semantics=("parallel",),
            vmem_limit_bytes=64 * 1024 * 1024,
        ),
    )(xt, w1q, b1c, w2b, b2c, fw1p, fb1c, fw2p, fb2c)


def kernel(x, w1b, b1c, w2b, b2c, fw1p, fb1c, fw2p, fb2c):
    n = x.shape[0]
    bn = _BN
    n_pad = ((n + bn - 1) // bn) * bn

    xt = jnp.transpose(x, (2, 3, 1, 0)).reshape(_IN_FEATS, n)
    xt = xt.astype(jnp.bfloat16)
    if n_pad != n:
        xt = jnp.pad(xt, ((0, 0), (0, n_pad - n)))

    c_, h_, w_ = np.meshgrid(np.arange(2), np.arange(14), np.arange(14),
                             indexing="ij")
    pm = np.zeros((_IN_FEATS, _IN_FEATS), np.float32)
    pm[(c_ * 196 + h_ * 14 + w_).ravel(),
       ((h_ * 14 + w_) * 2 + c_).ravel()] = 1.0
    w1q = jnp.dot(w1b, jnp.asarray(pm, jnp.bfloat16),
                  preferred_element_type=jnp.bfloat16)

    out_row = _forward_block(xt, w1q, b1c, w2b, b2c, fw1p, fb1c, fw2p, fb2c)

    return out_row[0, :n].reshape(n, 1).astype(x.dtype)


# speedup vs baseline: 1.0294x; 1.0294x over previous
import jax
import jax.numpy as jnp
import numpy as np
from jax.experimental import pallas as pl
from jax.experimental.pallas import tpu as pltpu

_BN = 2048
_IN_FEATS = 392
_C1_ROWS = 9 * 512
_MC = 512


def _net_kernel(x_ref, w1_ref, b1_ref, w2_ref, b2_ref,
                fw1_ref, fb1_ref, fw2_ref, fb2_ref, out_ref, p1_ref):
    f32 = jnp.float32

    xb = x_ref[...]

    for mc in range(512 // _MC):
        lo = mc * _MC
        p1 = jnp.dot(w1_ref[lo:lo + _MC, :], xb, preferred_element_type=f32)
        for t in range(1, 9):
            c = jnp.dot(w1_ref[t * 512 + lo:t * 512 + lo + _MC, :], xb,
                        preferred_element_type=f32)
            p1 = jnp.maximum(p1, c)
        p1 = jnp.maximum(p1 + b1_ref[lo:lo + _MC, :], 0.0)
        p1_ref[lo:lo + _MC, :] = p1.astype(jnp.bfloat16)

    c2 = jnp.dot(w2_ref[...], p1_ref[...],
                 preferred_element_type=f32) + b2_ref[...]
    p2 = jnp.maximum(jnp.maximum(c2[0:64, :], c2[64:128, :]),
                     jnp.maximum(c2[128:192, :], c2[192:256, :]))
    p2 = jnp.maximum(p2, 0.0)

    h = jnp.dot(fw1_ref[...], p2.astype(jnp.bfloat16),
                preferred_element_type=f32) + fb1_ref[...]
    h = jnp.maximum(h, 0.0)

    o8 = jnp.dot(fw2_ref[...], h.astype(jnp.bfloat16),
                 preferred_element_type=f32)
    o = o8[0:1, :] + fb2_ref[...]
    out_ref[...] = jnp.tanh(jnp.maximum(o, 0.0)).astype(out_ref.dtype)


def _forward_block(xt, w1q, b1c, w2b, b2c, fw1p, fb1c, fw2p, fb2c):
    bn = _BN
    m = xt.shape[1]
    grid = (m // bn,)
    in_specs = [
        pl.BlockSpec((_IN_FEATS, bn), lambda b: (0, b)),
        pl.BlockSpec((_C1_ROWS, _IN_FEATS), lambda b: (0, 0)),
        pl.BlockSpec((512, 1), lambda b: (0, 0)),
        pl.BlockSpec((256, 512), lambda b: (0, 0)),
        pl.BlockSpec((256, 1), lambda b: (0, 0)),
        pl.BlockSpec((256, 64), lambda b: (0, 0)),
        pl.BlockSpec((256, 1), lambda b: (0, 0)),
        pl.BlockSpec((8, 256), lambda b: (0, 0)),
        pl.BlockSpec((1, 1), lambda b: (0, 0)),
    ]
    out_specs = pl.BlockSpec((1, bn), lambda b: (0, b))

    return pl.pallas_call(
        _net_kernel,
        out_shape=jax.ShapeDtypeStruct((1, m), jnp.float32),
        grid_spec=pltpu.PrefetchScalarGridSpec(
            num_scalar_prefetch=0,
            grid=grid,
            in_specs=in_specs,
            out_specs=out_specs,
            scratch_shapes=[pltpu.VMEM((512, bn), jnp.bfloat16)],
        ),
        compiler_params=pltpu.CompilerParams(
            dimension_semantics=("parallel",),
            vmem_limit_bytes=64 * 1024 * 1024,
        ),
    )(xt, w1q, b1c, w2b, b2c, fw1p, fb1c, fw2p, fb2c)


def kernel(x, w1b, b1c, w2b, b2c, fw1p, fb1c, fw2p, fb2c):
    n = x.shape[0]
    bn = _BN
    n_pad = ((n + bn - 1) // bn) * bn

    xt = jnp.transpose(x, (2, 3, 1, 0)).reshape(_IN_FEATS, n)
    xt = xt.astype(jnp.bfloat16)
    if n_pad != n:
        xt = jnp.pad(xt, ((0, 0), (0, n_pad - n)))

    c_, h_, w_ = np.meshgrid(np.arange(2), np.arange(14), np.arange(14),
                             indexing="ij")
    pm = np.zeros((_IN_FEATS, _IN_FEATS), np.float32)
    pm[(c_ * 196 + h_ * 14 + w_).ravel(),
       ((h_ * 14 + w_) * 2 + c_).ravel()] = 1.0
    w1q = jnp.dot(w1b, jnp.asarray(pm, jnp.bfloat16),
                  preferred_element_type=jnp.bfloat16)

    out_row = _forward_block(xt, w1q, b1c, w2b, b2c, fw1p, fb1c, fw2p, fb2c)

    return out_row[0, :n].reshape(n, 1).astype(x.dtype)
